# Initial kernel scaffold; baseline (speedup 1.0000x reference)
import functools

import jax
import jax.numpy as jnp
from jax import lax
from jax.experimental import pallas as pl
from jax.experimental.pallas import tpu as pltpu
from jax.experimental.pallas import tpu_sc as plsc

N_NODES = 100016
N_EDGES = 3200512
N_GRAPHS = 2632

NC = 2
NS = 16
NW = NC * NS
L = 16

C = 2048
K = C // 128
CHUNKS = -(-N_EDGES // (NW * C))
EPW = CHUNKS * C
EPAD = EPW * NW
NPAD = 100096
SLICE = NPAD // NS


def _sc_body(src_hbm, dst_hbm, ea_hbm, x_hbm, wnn_hbm, bnn_hbm, out_hbm,
             x_v, src_v, ea_v, dst_v, msg_v, zero_v, wnn_v, bnn_v, acc_sh):
    cid = lax.axis_index("c")
    sid = lax.axis_index("s")
    wid = cid * NS + sid

    pltpu.sync_copy(x_hbm, x_v)
    pltpu.sync_copy(wnn_hbm, wnn_v)
    pltpu.sync_copy(bnn_hbm, bnn_v)
    w = wnn_v[...]
    b = bnn_v[...]

    def _zero(i, carry):
        zero_v[pl.ds(i * L, L)] = jnp.zeros((L,), jnp.float32)
        return carry
    lax.fori_loop(0, SLICE // L, _zero, 0)
    pltpu.sync_copy(zero_v, acc_sh.at[pl.ds(sid * SLICE, SLICE)])
    plsc.subcore_barrier()

    def _chunk(i, carry):
        ebase = wid * EPW + i * C
        rbase = (wid * EPW) // 128 + i * K
        pltpu.sync_copy(src_hbm.at[pl.ds(ebase, C)], src_v)
        pltpu.sync_copy(ea_hbm.at[pl.ds(ebase, C)], ea_v)
        pltpu.sync_copy(dst_hbm.at[pl.ds(rbase, K)], dst_v)

        def _row(k, c2):
            for g in range(8):
                off = k * 128 + g * L
                s = src_v[pl.ds(off, L)]
                xg = plsc.load_gather(x_v, [s])
                a = ea_v[pl.ds(off, L)]
                msg_v[pl.ds(off, L)] = xg * (a * w + b)
            return c2
        lax.fori_loop(0, K, _row, 0)

        def _scat(k, c2):
            pltpu.sync_copy(msg_v.at[pl.ds(k * 128, 128)],
                            acc_sh.at[dst_v.at[k]], add=True)
            return c2
        lax.fori_loop(0, K, _scat, 0)
        return carry
    lax.fori_loop(0, CHUNKS, _chunk, 0)

    plsc.subcore_barrier()
    pltpu.sync_copy(acc_sh.at[pl.ds(sid * SLICE, SLICE)],
                    out_hbm.at[cid, pl.ds(sid * SLICE, SLICE)])


@jax.jit
def _sc_scatter(srcp, dstp, eap, xflat, wnn_v, bnn_v):
    mesh = plsc.VectorSubcoreMesh(core_axis_name="c", subcore_axis_name="s")
    return pl.kernel(
        _sc_body,
        out_type=jax.ShapeDtypeStruct((NC, NPAD), jnp.float32),
        mesh=mesh,
        scratch_types=[
            pltpu.VMEM((N_NODES,), jnp.float32),
            pltpu.VMEM((C,), jnp.int32),
            pltpu.VMEM((C,), jnp.float32),
            pltpu.VMEM((K, 128), jnp.int32),
            pltpu.VMEM((C,), jnp.float32),
            pltpu.VMEM((SLICE,), jnp.float32),
            pltpu.VMEM((L,), jnp.float32),
            pltpu.VMEM((L,), jnp.float32),
            pltpu.VMEM_SHARED((NPAD,), jnp.float32),
        ],
    )(srcp, dstp, eap, xflat, wnn_v, bnn_v)


def _tc_body(p0, p1, x2d, root, cb, w1, b1, w2, b2, w3, b3, out):
    nodes = p0[...] + p1[...] + x2d[...] * root[...] + cb[...]
    h = lax.dot_general(nodes, w1[...], (((1,), (0,)), ((), ())),
                        preferred_element_type=jnp.float32) + b1[...]
    h = jnp.where(h > 0, h, 0.01 * h)
    h = lax.dot_general(h, w2[...], (((1,), (0,)), ((), ())),
                        preferred_element_type=jnp.float32) + b2[...]
    h = jnp.where(h > 0, h, 0.01 * h)
    h = lax.dot_general(h, w3[...], (((1,), (0,)), ((), ())),
                        preferred_element_type=jnp.float32) + b3[...]
    h = jnp.where(h > 0, h, 0.01 * h)
    m = jnp.max(h, axis=-1, keepdims=True)
    e = jnp.exp(h - m)
    out[...] = e / jnp.sum(e, axis=-1, keepdims=True)


@jax.jit
def _tc_tail(p0, p1, x2d, root, cb, w1, b1, w2, b2, w3, b3):
    return pl.pallas_call(
        _tc_body,
        out_shape=jax.ShapeDtypeStruct((N_GRAPHS, 12), jnp.float32),
    )(p0, p1, x2d, root, cb, w1, b1, w2, b2, w3, b3)


def kernel(x, edge_index, edge_attr, batch, Wnn, bnn, root, conv_bias,
           W1, b1, W2, b2, W3, b3):
    del batch
    pad = EPAD - N_EDGES
    src = edge_index[0]
    dst = edge_index[1]
    ea = edge_attr[:, 0]
    srcp = jnp.concatenate([src, jnp.zeros((pad,), jnp.int32)])
    dstp = jnp.concatenate([dst, jnp.full((pad,), N_NODES, jnp.int32)])
    dstp = dstp.reshape(EPAD // 128, 128)
    eap = jnp.concatenate([ea, jnp.zeros((pad,), jnp.float32)])
    xflat = x[:, 0]
    wnn_v = jnp.full((L,), Wnn[0, 0], jnp.float32)
    bnn_v = jnp.full((L,), bnn[0], jnp.float32)

    partial = _sc_scatter(srcp, dstp, eap, xflat, wnn_v, bnn_v)

    p0 = partial[0, :N_NODES].reshape(N_GRAPHS, 38)
    p1 = partial[1, :N_NODES].reshape(N_GRAPHS, 38)
    x2d = xflat.reshape(N_GRAPHS, 38)
    return _tc_tail(p0, p1, x2d,
                    root.reshape(1, 1), conv_bias.reshape(1, 1),
                    W1, b1.reshape(1, 4), W2, b2.reshape(1, 4),
                    W3, b3.reshape(1, 12))


# speedup vs baseline: 50.3104x; 50.3104x over previous
import functools

import jax
import jax.numpy as jnp
from jax import lax
from jax.experimental import pallas as pl
from jax.experimental.pallas import tpu as pltpu
from jax.experimental.pallas import tpu_sc as plsc

N_NODES = 100016
N_EDGES = 3200512
N_GRAPHS = 2632

NC = 2
NS = 16
NW = NC * NS
L = 16

C = 2048
K = C // 128
CHUNKS = -(-N_EDGES // (NW * C))
EPW = CHUNKS * C
EPAD = EPW * NW
NPAD = 100096
SLICE = NPAD // NS


def _sc_body(src_hbm, dst_hbm, ea_hbm, x_hbm, wnn_hbm, bnn_hbm, out_hbm,
             x_v, src_v, ea_v, dst_v, msg_v, zero_v, wnn_v, bnn_v, acc_sh):
    cid = lax.axis_index("c")
    sid = lax.axis_index("s")
    wid = cid * NS + sid

    pltpu.sync_copy(x_hbm, x_v)
    pltpu.sync_copy(wnn_hbm, wnn_v)
    pltpu.sync_copy(bnn_hbm, bnn_v)
    w = wnn_v[...]
    b = bnn_v[...]

    def _zero(i, carry):
        zero_v[pl.ds(i * L, L)] = jnp.zeros((L,), jnp.float32)
        return carry
    lax.fori_loop(0, SLICE // L, _zero, 0)
    pltpu.sync_copy(zero_v, acc_sh.at[pl.ds(sid * SLICE, SLICE)])
    plsc.subcore_barrier()

    def _chunk(i, carry):
        ebase = pl.multiple_of(wid * EPW + i * C, 8)
        rbase = pl.multiple_of(wid * (EPW // 128) + i * K, 8)
        pltpu.sync_copy(src_hbm.at[pl.ds(ebase, C)], src_v)
        pltpu.sync_copy(ea_hbm.at[pl.ds(ebase, C)], ea_v)
        pltpu.sync_copy(dst_hbm.at[pl.ds(rbase, K)], dst_v)

        def _row(k, c2):
            for g in range(8):
                off = k * 128 + g * L
                s = src_v[pl.ds(off, L)]
                xg = plsc.load_gather(x_v, [s])
                a = ea_v[pl.ds(off, L)]
                msg_v[pl.ds(off, L)] = xg * (a * w + b)
            return c2
        lax.fori_loop(0, K, _row, 0)

        def _scat(k, c2):
            pltpu.sync_copy(msg_v.at[pl.ds(k * 128, 128)],
                            acc_sh.at[dst_v.at[k]], add=True)
            return c2
        lax.fori_loop(0, K, _scat, 0)
        return carry
    lax.fori_loop(0, CHUNKS, _chunk, 0)

    plsc.subcore_barrier()
    obase = pl.multiple_of(cid * NPAD + sid * SLICE, 8)
    pltpu.sync_copy(acc_sh.at[pl.ds(sid * SLICE, SLICE)], zero_v)
    pltpu.sync_copy(zero_v, out_hbm.at[pl.ds(obase, SLICE)])


@jax.jit
def _sc_scatter(srcp, dstp, eap, xflat, wnn_v, bnn_v):
    mesh = plsc.VectorSubcoreMesh(core_axis_name="c", subcore_axis_name="s")
    return pl.kernel(
        _sc_body,
        out_type=jax.ShapeDtypeStruct((NC * NPAD,), jnp.float32),
        mesh=mesh,
        compiler_params=pltpu.CompilerParams(needs_layout_passes=False),
        scratch_types=[
            pltpu.VMEM((N_NODES,), jnp.float32),
            pltpu.VMEM((C,), jnp.int32),
            pltpu.VMEM((C,), jnp.float32),
            pltpu.VMEM((K, 128), jnp.int32),
            pltpu.VMEM((C,), jnp.float32),
            pltpu.VMEM((SLICE,), jnp.float32),
            pltpu.VMEM((L,), jnp.float32),
            pltpu.VMEM((L,), jnp.float32),
            pltpu.VMEM_SHARED((NPAD,), jnp.float32),
        ],
    )(srcp, dstp, eap, xflat, wnn_v, bnn_v)


def _tc_body(p0, p1, x2d, root, cb, w1, b1, w2, b2, w3, b3, out):
    nodes = p0[...] + p1[...] + x2d[...] * root[...] + cb[...]
    h = lax.dot_general(nodes, w1[...], (((1,), (0,)), ((), ())),
                        preferred_element_type=jnp.float32) + b1[...]
    h = jnp.where(h > 0, h, 0.01 * h)
    h = lax.dot_general(h, w2[...], (((1,), (0,)), ((), ())),
                        preferred_element_type=jnp.float32) + b2[...]
    h = jnp.where(h > 0, h, 0.01 * h)
    h = lax.dot_general(h, w3[...], (((1,), (0,)), ((), ())),
                        preferred_element_type=jnp.float32) + b3[...]
    h = jnp.where(h > 0, h, 0.01 * h)
    m = jnp.max(h, axis=-1, keepdims=True)
    e = jnp.exp(h - m)
    out[...] = e / jnp.sum(e, axis=-1, keepdims=True)


@jax.jit
def _tc_tail(p0, p1, x2d, root, cb, w1, b1, w2, b2, w3, b3):
    return pl.pallas_call(
        _tc_body,
        out_shape=jax.ShapeDtypeStruct((N_GRAPHS, 12), jnp.float32),
    )(p0, p1, x2d, root, cb, w1, b1, w2, b2, w3, b3)


def kernel(x, edge_index, edge_attr, batch, Wnn, bnn, root, conv_bias,
           W1, b1, W2, b2, W3, b3):
    del batch
    pad = EPAD - N_EDGES
    src = edge_index[0]
    dst = edge_index[1]
    ea = edge_attr[:, 0]
    srcp = jnp.concatenate([src, jnp.zeros((pad,), jnp.int32)])
    dstp = jnp.concatenate([dst, jnp.full((pad,), N_NODES, jnp.int32)])
    dstp = dstp.reshape(EPAD // 128, 128)
    eap = jnp.concatenate([ea, jnp.zeros((pad,), jnp.float32)])
    xflat = x[:, 0]
    wnn_v = jnp.full((L,), Wnn[0, 0], jnp.float32)
    bnn_v = jnp.full((L,), bnn[0], jnp.float32)

    partial = _sc_scatter(srcp, dstp, eap, xflat, wnn_v, bnn_v)

    p0 = partial[:N_NODES].reshape(N_GRAPHS, 38)
    p1 = partial[NPAD:NPAD + N_NODES].reshape(N_GRAPHS, 38)
    x2d = xflat.reshape(N_GRAPHS, 38)
    return _tc_tail(p0, p1, x2d,
                    root.reshape(1, 1), conv_bias.reshape(1, 1),
                    W1, b1.reshape(1, 4), W2, b2.reshape(1, 4),
                    W3, b3.reshape(1, 12))


# speedup vs baseline: 53.4139x; 1.0617x over previous
import functools

import jax
import jax.numpy as jnp
from jax import lax
from jax.experimental import pallas as pl
from jax.experimental.pallas import tpu as pltpu
from jax.experimental.pallas import tpu_sc as plsc

N_NODES = 100016
N_EDGES = 3200512
N_GRAPHS = 2632

NC = 2
NS = 16
NW = NC * NS
L = 16

C = 2048
K = C // 128
MAIN_CHUNKS = 48
EPW = MAIN_CHUNKS * C
E_MAIN = EPW * NW
E_TAIL = N_EDGES - E_MAIN
TAILPAD = NW * C
NPAD = 100096
SLICE = NPAD // NS
NBUF = 4


def _sc_body(src_hbm, dst_hbm, ea_hbm, tsrc_hbm, tdst_hbm, tea_hbm,
             x_hbm, wnn_hbm, bnn_hbm, out_hbm,
             src_v0, src_v1, src_v2, src_v3, ea_v0, ea_v1, ea_v2, ea_v3,
             dst_v0, dst_v1, dst_v2, dst_v3, msg_v0, msg_v1, msg_v2, msg_v3,
             xg_v0, xg_v1, xg_v2, xg_v3,
             zero_v, wnn_v, bnn_v, acc_sh, sem_in, sem_g, sem_scat):
    src_v = (src_v0, src_v1, src_v2, src_v3)
    ea_v = (ea_v0, ea_v1, ea_v2, ea_v3)
    dst_v = (dst_v0, dst_v1, dst_v2, dst_v3)
    msg_v = (msg_v0, msg_v1, msg_v2, msg_v3)
    xg_v = (xg_v0, xg_v1, xg_v2, xg_v3)
    cid = lax.axis_index("c")
    sid = lax.axis_index("s")
    wid = cid * NS + sid

    pltpu.sync_copy(wnn_hbm, wnn_v)
    pltpu.sync_copy(bnn_hbm, bnn_v)
    w = wnn_v[...]
    b = bnn_v[...]

    def _zero(i, carry):
        zero_v[pl.ds(i * L, L)] = jnp.zeros((L,), jnp.float32)
        return carry
    lax.fori_loop(0, SLICE // L, _zero, 0)
    pltpu.sync_copy(zero_v, acc_sh.at[pl.ds(sid * SLICE, SLICE)])
    plsc.subcore_barrier()

    ebase0 = wid * EPW

    def fire_in(ic, bb):
        eb = pl.multiple_of(ebase0 + ic * C, 8)
        pltpu.async_copy(src_hbm.at[pl.ds(eb, C)], src_v[bb], sem_in.at[bb])
        pltpu.async_copy(ea_hbm.at[pl.ds(eb, C)], ea_v[bb], sem_in.at[bb])
        pltpu.async_copy(dst_hbm.at[pl.ds(eb, C)], dst_v[bb], sem_in.at[bb])

    def wait_in(ic, bb):
        eb = pl.multiple_of(ebase0 + ic * C, 8)
        pltpu.make_async_copy(src_hbm.at[pl.ds(eb, C)], src_v[bb],
                              sem_in.at[bb]).wait()
        pltpu.make_async_copy(ea_hbm.at[pl.ds(eb, C)], ea_v[bb],
                              sem_in.at[bb]).wait()
        pltpu.make_async_copy(dst_hbm.at[pl.ds(eb, C)], dst_v[bb],
                              sem_in.at[bb]).wait()

    def fire_gather(bb):
        pltpu.async_copy(x_hbm.at[src_v[bb]], xg_v[bb], sem_g.at[bb])

    def wait_gather(bb):
        pltpu.make_async_copy(x_hbm.at[src_v[bb]], xg_v[bb],
                              sem_g.at[bb]).wait()

    def compute(bb):
        def _row(k, c2):
            for g in range(8):
                off = k * 128 + g * L
                xg = xg_v[bb][pl.ds(off, L)]
                a = ea_v[bb][pl.ds(off, L)]
                msg_v[bb][pl.ds(off, L)] = xg * (a * w + b)
            return c2
        lax.fori_loop(0, K, _row, 0)

    def fire_scat(bb):
        pltpu.async_copy(msg_v[bb], acc_sh.at[dst_v[bb]],
                         sem_scat.at[bb], add=True)

    def drain_scat(bb):
        pltpu.make_async_copy(msg_v[bb], acc_sh.at[dst_v[bb]],
                              sem_scat.at[bb]).wait()

    fire_in(0, 0)
    fire_in(1, 1)

    def _iter(i, carry):
        k0 = i * NBUF
        for d in range(NBUF):
            ic = k0 + d
            bb = d
            pb = (d - 1) % NBUF
            ppb = (d - 2) % NBUF
            wait_in(ic, bb)
            fire_gather(bb)

            @pl.when(ic >= 1)
            def _():
                wait_gather(pb)
            if True:
                @pl.when(ic >= 1)
                def _():
                    compute(pb)

                @pl.when(ic >= 2)
                def _():
                    drain_scat(ppb)

                @pl.when(ic >= 1)
                def _():
                    fire_scat(pb)

                @pl.when(ic + 2 < MAIN_CHUNKS)
                def _():
                    fire_in(ic + 2, ppb)
        return carry
    lax.fori_loop(0, MAIN_CHUNKS // NBUF, _iter, 0)

    lb = (MAIN_CHUNKS - 1) % NBUF
    wait_gather(lb)
    compute(lb)
    drain_scat((MAIN_CHUNKS - 2) % NBUF)
    fire_scat(lb)
    drain_scat(lb)

    tb = pl.multiple_of(wid * C, 8)
    pltpu.sync_copy(tsrc_hbm.at[pl.ds(tb, C)], src_v[0])
    pltpu.sync_copy(tea_hbm.at[pl.ds(tb, C)], ea_v[0])
    pltpu.sync_copy(tdst_hbm.at[pl.ds(tb, C)], dst_v[0])
    fire_gather(0)
    wait_gather(0)
    compute(0)
    fire_scat(0)
    drain_scat(0)

    plsc.subcore_barrier()
    obase = pl.multiple_of(cid * NPAD + sid * SLICE, 8)
    pltpu.sync_copy(acc_sh.at[pl.ds(sid * SLICE, SLICE)], zero_v)
    pltpu.sync_copy(zero_v, out_hbm.at[pl.ds(obase, SLICE)])


@jax.jit
def _sc_scatter(src, dst, ea, tsrc, tdst, tea, xflat, wnn_v, bnn_v):
    mesh = plsc.VectorSubcoreMesh(core_axis_name="c", subcore_axis_name="s")
    return pl.kernel(
        _sc_body,
        out_type=jax.ShapeDtypeStruct((NC * NPAD,), jnp.float32),
        mesh=mesh,
        compiler_params=pltpu.CompilerParams(needs_layout_passes=False),
        scratch_types=[
            pltpu.VMEM((C,), jnp.int32),
            pltpu.VMEM((C,), jnp.int32),
            pltpu.VMEM((C,), jnp.int32),
            pltpu.VMEM((C,), jnp.int32),
            pltpu.VMEM((C,), jnp.float32),
            pltpu.VMEM((C,), jnp.float32),
            pltpu.VMEM((C,), jnp.float32),
            pltpu.VMEM((C,), jnp.float32),
            pltpu.VMEM((C,), jnp.int32),
            pltpu.VMEM((C,), jnp.int32),
            pltpu.VMEM((C,), jnp.int32),
            pltpu.VMEM((C,), jnp.int32),
            pltpu.VMEM((C,), jnp.float32),
            pltpu.VMEM((C,), jnp.float32),
            pltpu.VMEM((C,), jnp.float32),
            pltpu.VMEM((C,), jnp.float32),
            pltpu.VMEM((C,), jnp.float32),
            pltpu.VMEM((C,), jnp.float32),
            pltpu.VMEM((C,), jnp.float32),
            pltpu.VMEM((C,), jnp.float32),
            pltpu.VMEM((SLICE,), jnp.float32),
            pltpu.VMEM((L,), jnp.float32),
            pltpu.VMEM((L,), jnp.float32),
            pltpu.VMEM_SHARED((NPAD,), jnp.float32),
            pltpu.SemaphoreType.DMA((NBUF,)),
            pltpu.SemaphoreType.DMA((NBUF,)),
            pltpu.SemaphoreType.DMA((NBUF,)),
        ],
    )(src, dst, ea, tsrc, tdst, tea, xflat, wnn_v, bnn_v)


def _tc_body(p0, p1, x2d, root, cb, w1, b1, w2, b2, w3, b3, out):
    nodes = p0[...] + p1[...] + x2d[...] * root[...] + cb[...]
    h = lax.dot_general(nodes, w1[...], (((1,), (0,)), ((), ())),
                        preferred_element_type=jnp.float32) + b1[...]
    h = jnp.where(h > 0, h, 0.01 * h)
    h = lax.dot_general(h, w2[...], (((1,), (0,)), ((), ())),
                        preferred_element_type=jnp.float32) + b2[...]
    h = jnp.where(h > 0, h, 0.01 * h)
    h = lax.dot_general(h, w3[...], (((1,), (0,)), ((), ())),
                        preferred_element_type=jnp.float32) + b3[...]
    h = jnp.where(h > 0, h, 0.01 * h)
    m = jnp.max(h, axis=-1, keepdims=True)
    e = jnp.exp(h - m)
    out[...] = e / jnp.sum(e, axis=-1, keepdims=True)


@jax.jit
def _tc_tail(p0, p1, x2d, root, cb, w1, b1, w2, b2, w3, b3):
    return pl.pallas_call(
        _tc_body,
        out_shape=jax.ShapeDtypeStruct((N_GRAPHS, 12), jnp.float32),
    )(p0, p1, x2d, root, cb, w1, b1, w2, b2, w3, b3)


def kernel(x, edge_index, edge_attr, batch, Wnn, bnn, root, conv_bias,
           W1, b1, W2, b2, W3, b3):
    del batch
    src = edge_index[0]
    dst = edge_index[1]
    ea = edge_attr[:, 0]
    npad = TAILPAD - E_TAIL
    tsrc = jnp.concatenate([src[E_MAIN:], jnp.zeros((npad,), jnp.int32)])
    tdst = jnp.concatenate([dst[E_MAIN:],
                            jnp.full((npad,), N_NODES, jnp.int32)])
    tea = jnp.concatenate([ea[E_MAIN:], jnp.zeros((npad,), jnp.float32)])
    xflat = x[:, 0]
    wnn_v = jnp.full((L,), Wnn[0, 0], jnp.float32)
    bnn_v = jnp.full((L,), bnn[0], jnp.float32)

    partial = _sc_scatter(src, dst, ea, tsrc, tdst, tea,
                          xflat, wnn_v, bnn_v)

    p0 = partial[:N_NODES].reshape(N_GRAPHS, 38)
    p1 = partial[NPAD:NPAD + N_NODES].reshape(N_GRAPHS, 38)
    x2d = xflat.reshape(N_GRAPHS, 38)
    return _tc_tail(p0, p1, x2d,
                    root.reshape(1, 1), conv_bias.reshape(1, 1),
                    W1, b1.reshape(1, 4), W2, b2.reshape(1, 4),
                    W3, b3.reshape(1, 12))


# speedup vs baseline: 200.8667x; 3.7606x over previous
import functools

import jax
import jax.numpy as jnp
from jax import lax
from jax.experimental import pallas as pl
from jax.experimental.pallas import tpu as pltpu
from jax.experimental.pallas import tpu_sc as plsc

N_NODES = 100016
N_EDGES = 3200512
N_GRAPHS = 2632

NC = 2
NS = 16
NW = NC * NS
L = 16

C = 2048
K = C // 128
MAIN_CHUNKS = 48
EPW = MAIN_CHUNKS * C
E_MAIN = EPW * NW
E_TAIL = N_EDGES - E_MAIN
TAILPAD = NW * C
NPAD = 100096
SLICE = NPAD // NS
XLAST = N_NODES - (NS - 1) * SLICE
NBUF = 4


def _sc_body(ei_hbm, ea2_hbm, tei_hbm, tea_hbm,
             x_hbm, wnn_hbm, bnn_hbm, out_hbm,
             exc_v0, exc_v1, exc_v2, exc_v3, ea_v0, ea_v1, ea_v2, ea_v3,
             dst_v0, dst_v1, dst_v2, dst_v3, msg_v0, msg_v1, msg_v2, msg_v3,
             xg_v0, xg_v1, xg_v2, xg_v3, sf_v0, sf_v1, sf_v2, sf_v3,
             zero_v, wnn_v, bnn_v, acc_sh, x_sh, sem_in, sem_g, sem_scat):
    exc_v = (exc_v0, exc_v1, exc_v2, exc_v3)
    ea_v = (ea_v0, ea_v1, ea_v2, ea_v3)
    dst_v = (dst_v0, dst_v1, dst_v2, dst_v3)
    msg_v = (msg_v0, msg_v1, msg_v2, msg_v3)
    xg_v = (xg_v0, xg_v1, xg_v2, xg_v3)
    sf_v = (sf_v0, sf_v1, sf_v2, sf_v3)
    cid = lax.axis_index("c")
    sid = lax.axis_index("s")
    wid = cid * NS + sid

    pltpu.sync_copy(wnn_hbm, wnn_v)
    pltpu.sync_copy(bnn_hbm, bnn_v)
    w = wnn_v[...]
    b = bnn_v[...]

    def _zero(i, carry):
        zero_v[pl.ds(i * L, L)] = jnp.zeros((L,), jnp.float32)
        return carry
    lax.fori_loop(0, SLICE // L, _zero, 0)
    pltpu.sync_copy(zero_v, acc_sh.at[pl.ds(sid * SLICE, SLICE)])

    xoff = pl.multiple_of(sid * SLICE, 8)

    @pl.when(sid < NS - 1)
    def _():
        pltpu.sync_copy(x_hbm.at[pl.ds(xoff, SLICE)], zero_v)
        pltpu.sync_copy(zero_v, x_sh.at[pl.ds(xoff, SLICE)])

    @pl.when(sid == NS - 1)
    def _():
        pltpu.sync_copy(x_hbm.at[pl.ds(xoff, XLAST)],
                        zero_v.at[pl.ds(0, XLAST)])
        pltpu.sync_copy(zero_v.at[pl.ds(0, XLAST)],
                        x_sh.at[pl.ds(xoff, XLAST)])
    plsc.subcore_barrier()

    ebase0 = wid * EPW

    def fire_in(ic, bb):
        eb = pl.multiple_of(ebase0 + ic * C, 8)
        pltpu.async_copy(ei_hbm.at[:, pl.ds(eb, C)], exc_v[bb], sem_in.at[bb])
        pltpu.async_copy(ea2_hbm.at[0, pl.ds(eb, C)], ea_v[bb],
                         sem_in.at[bb])

    def wait_in(ic, bb):
        eb = pl.multiple_of(ebase0 + ic * C, 8)
        pltpu.make_async_copy(ei_hbm.at[:, pl.ds(eb, C)], exc_v[bb],
                              sem_in.at[bb]).wait()
        pltpu.make_async_copy(ea2_hbm.at[0, pl.ds(eb, C)], ea_v[bb],
                              sem_in.at[bb]).wait()

    def repack(bb):
        def _row(k, c2):
            for g in range(8):
                off = k * 128 + g * L
                sf_v[bb][pl.ds(off, L)] = exc_v[bb][0, pl.ds(off, L)]
                dst_v[bb][pl.ds(off, L)] = exc_v[bb][1, pl.ds(off, L)]
            return c2
        lax.fori_loop(0, K, _row, 0)

    def fire_gather(bb):
        pltpu.async_copy(x_sh.at[sf_v[bb]], xg_v[bb], sem_g.at[bb])

    def wait_gather(bb):
        pltpu.make_async_copy(x_sh.at[sf_v[bb]], xg_v[bb],
                              sem_g.at[bb]).wait()

    def compute(bb):
        def _row(k, c2):
            for g in range(8):
                off = k * 128 + g * L
                xg = xg_v[bb][pl.ds(off, L)]
                a = ea_v[bb][pl.ds(off, L)]
                msg_v[bb][pl.ds(off, L)] = xg * (a * w + b)
            return c2
        lax.fori_loop(0, K, _row, 0)

    def fire_scat(bb):
        pltpu.async_copy(msg_v[bb], acc_sh.at[dst_v[bb]],
                         sem_scat.at[bb], add=True)

    def drain_scat(bb):
        pltpu.make_async_copy(msg_v[bb], acc_sh.at[dst_v[bb]],
                              sem_scat.at[bb]).wait()

    fire_in(0, 0)
    fire_in(1, 1)

    def _iter(i, carry):
        k0 = i * NBUF
        for d in range(NBUF):
            ic = k0 + d
            bb = d
            pb = (d - 1) % NBUF
            ppb = (d - 2) % NBUF
            wait_in(ic, bb)
            repack(bb)
            fire_gather(bb)

            @pl.when(ic >= 1)
            def _():
                wait_gather(pb)
                compute(pb)

            @pl.when(ic >= 2)
            def _():
                drain_scat(ppb)

            @pl.when(ic >= 1)
            def _():
                fire_scat(pb)

            @pl.when(ic + 2 < MAIN_CHUNKS)
            def _():
                fire_in(ic + 2, ppb)
        return carry
    lax.fori_loop(0, MAIN_CHUNKS // NBUF, _iter, 0)

    lb = (MAIN_CHUNKS - 1) % NBUF
    wait_gather(lb)
    compute(lb)
    drain_scat((MAIN_CHUNKS - 2) % NBUF)
    fire_scat(lb)
    drain_scat(lb)

    tb = pl.multiple_of(wid * C, 8)
    pltpu.sync_copy(tei_hbm.at[:, pl.ds(tb, C)], exc_v[0])
    pltpu.sync_copy(tea_hbm.at[pl.ds(tb, C)], ea_v[0])
    repack(0)
    fire_gather(0)
    wait_gather(0)
    compute(0)
    fire_scat(0)
    drain_scat(0)

    plsc.subcore_barrier()
    obase = pl.multiple_of(cid * NPAD + sid * SLICE, 8)
    pltpu.sync_copy(acc_sh.at[pl.ds(sid * SLICE, SLICE)], zero_v)
    pltpu.sync_copy(zero_v, out_hbm.at[pl.ds(obase, SLICE)])


@jax.jit
def _sc_scatter(ei, ea, tei, tea, xflat, wnn_v, bnn_v):
    mesh = plsc.VectorSubcoreMesh(core_axis_name="c", subcore_axis_name="s")
    return pl.kernel(
        _sc_body,
        out_type=jax.ShapeDtypeStruct((NC * NPAD,), jnp.float32),
        mesh=mesh,
        compiler_params=pltpu.CompilerParams(needs_layout_passes=False),
        scratch_types=[
            pltpu.VMEM((2, C), jnp.int32),
            pltpu.VMEM((2, C), jnp.int32),
            pltpu.VMEM((2, C), jnp.int32),
            pltpu.VMEM((2, C), jnp.int32),
            pltpu.VMEM((C,), jnp.float32),
            pltpu.VMEM((C,), jnp.float32),
            pltpu.VMEM((C,), jnp.float32),
            pltpu.VMEM((C,), jnp.float32),
            pltpu.VMEM((C,), jnp.int32),
            pltpu.VMEM((C,), jnp.int32),
            pltpu.VMEM((C,), jnp.int32),
            pltpu.VMEM((C,), jnp.int32),
            pltpu.VMEM((C,), jnp.float32),
            pltpu.VMEM((C,), jnp.float32),
            pltpu.VMEM((C,), jnp.float32),
            pltpu.VMEM((C,), jnp.float32),
            pltpu.VMEM((C,), jnp.float32),
            pltpu.VMEM((C,), jnp.float32),
            pltpu.VMEM((C,), jnp.float32),
            pltpu.VMEM((C,), jnp.float32),
            pltpu.VMEM((C,), jnp.int32),
            pltpu.VMEM((C,), jnp.int32),
            pltpu.VMEM((C,), jnp.int32),
            pltpu.VMEM((C,), jnp.int32),
            pltpu.VMEM((SLICE,), jnp.float32),
            pltpu.VMEM((L,), jnp.float32),
            pltpu.VMEM((L,), jnp.float32),
            pltpu.VMEM_SHARED((NPAD,), jnp.float32),
            pltpu.VMEM_SHARED((NPAD,), jnp.float32),
            pltpu.SemaphoreType.DMA((NBUF,)),
            pltpu.SemaphoreType.DMA((NBUF,)),
            pltpu.SemaphoreType.DMA((NBUF,)),
        ],
    )(ei, ea, tei, tea, xflat, wnn_v, bnn_v)


def _tc_body(p0, p1, x2d, root, cb, w1, b1, w2, b2, w3, b3, out):
    nodes = p0[...] + p1[...] + x2d[...] * root[...] + cb[...]
    h = lax.dot_general(nodes, w1[...], (((1,), (0,)), ((), ())),
                        preferred_element_type=jnp.float32) + b1[...]
    h = jnp.where(h > 0, h, 0.01 * h)
    h = lax.dot_general(h, w2[...], (((1,), (0,)), ((), ())),
                        preferred_element_type=jnp.float32) + b2[...]
    h = jnp.where(h > 0, h, 0.01 * h)
    h = lax.dot_general(h, w3[...], (((1,), (0,)), ((), ())),
                        preferred_element_type=jnp.float32) + b3[...]
    h = jnp.where(h > 0, h, 0.01 * h)
    m = jnp.max(h, axis=-1, keepdims=True)
    e = jnp.exp(h - m)
    out[...] = e / jnp.sum(e, axis=-1, keepdims=True)


@jax.jit
def _tc_tail(p0, p1, x2d, root, cb, w1, b1, w2, b2, w3, b3):
    return pl.pallas_call(
        _tc_body,
        out_shape=jax.ShapeDtypeStruct((N_GRAPHS, 12), jnp.float32),
    )(p0, p1, x2d, root, cb, w1, b1, w2, b2, w3, b3)


def kernel(x, edge_index, edge_attr, batch, Wnn, bnn, root, conv_bias,
           W1, b1, W2, b2, W3, b3):
    del batch
    npad = TAILPAD - E_TAIL
    pad_ei = jnp.stack([jnp.zeros((npad,), jnp.int32),
                        jnp.full((npad,), N_NODES, jnp.int32)])
    tei = jnp.concatenate([edge_index[:, E_MAIN:], pad_ei], axis=1)
    tea = jnp.concatenate([edge_attr[E_MAIN:, 0],
                           jnp.zeros((npad,), jnp.float32)])
    xflat = x.reshape(N_NODES)
    wnn_v = jnp.full((L,), Wnn[0, 0], jnp.float32)
    bnn_v = jnp.full((L,), bnn[0], jnp.float32)

    partial = _sc_scatter(edge_index, edge_attr.T, tei, tea,
                          xflat, wnn_v, bnn_v)

    p0 = partial[:N_NODES].reshape(N_GRAPHS, 38)
    p1 = partial[NPAD:NPAD + N_NODES].reshape(N_GRAPHS, 38)
    x2d = xflat.reshape(N_GRAPHS, 38)
    return _tc_tail(p0, p1, x2d,
                    root.reshape(1, 1), conv_bias.reshape(1, 1),
                    W1, b1.reshape(1, 4), W2, b2.reshape(1, 4),
                    W3, b3.reshape(1, 12))
